# baseline (speedup 1.0000x reference)
import functools

import jax
import jax.numpy as jnp
from jax import lax
from jax.experimental import pallas as pl
from jax.experimental.pallas import tpu as pltpu
from jax.experimental.pallas import tpu_sc as plsc

NC = 2
NS = 16
NW = NC * NS
L = 16


@functools.lru_cache(maxsize=None)
def _build(B, D, K, N, DW):
    assert B % NW == 0, B
    assert D % L == 0 and DW >= D, D
    assert K % 2 == 0 and (K // 2) <= 128 and K % 8 == 0 and K >= L, K
    bpw = B // NW
    kh = K // 2
    ngroups = (K + L - 1) // L
    kpad = ngroups * L
    ndc = D // L

    mesh = plsc.VectorSubcoreMesh(core_axis_name="c", subcore_axis_name="s")

    @functools.partial(
        pl.kernel,
        mesh=mesh,
        compiler_params=pltpu.CompilerParams(use_tc_tiling_on_sc=False),
        out_type=jax.ShapeDtypeStruct((B, kpad), jnp.float32),
        scratch_types=[
            pltpu.VMEM((bpw, 2, kh), jnp.int32),
            pltpu.VMEM((bpw, D), jnp.float32),
            pltpu.VMEM((2, kpad, DW), jnp.float32),
            pltpu.VMEM((bpw, kpad), jnp.float32),
            pltpu.SemaphoreType.DMA,
            pltpu.SemaphoreType.DMA,
        ],
    )
    def sc_kernel(h_hbm, ids_hbm, w_hbm, out_hbm, idx_v, h_v, emb, out_v,
                  sem0, sem1):
        wid = lax.axis_index("s") * NC + lax.axis_index("c")
        base = wid * bpw

        pltpu.sync_copy(ids_hbm.at[pl.ds(base, bpw)], idx_v)
        pltpu.sync_copy(h_hbm.at[pl.ds(base, bpw)], h_v)

        sems = (sem0, sem1)

        def fire(b, slot):
            pltpu.async_copy(w_hbm.at[idx_v.at[b, 0]],
                             emb.at[slot, pl.ds(0, kh)], sems[slot])
            pltpu.async_copy(w_hbm.at[idx_v.at[b, 1]],
                             emb.at[slot, pl.ds(kh, kh)], sems[slot])

        def drain(slot):
            pltpu.make_async_copy(w_hbm.at[pl.ds(0, K)],
                                  emb.at[slot, pl.ds(0, K)],
                                  sems[slot]).wait()

        lane = lax.iota(jnp.int32, L)
        perms = [lane ^ (1 << i) for i in range(4)]
        masks = [(lane & (1 << i)) == 0 for i in range(4)]

        dnums = lax.GatherDimensionNumbers(
            offset_dims=(), collapsed_slice_dims=(0,), start_index_map=(0,))

        def shuffle(x, perm):
            return lax.gather(x, perm[:, None], dimension_numbers=dnums,
                              slice_sizes=(1,),
                              mode=lax.GatherScatterMode.PROMISE_IN_BOUNDS)

        def combine(x, y, lvl):
            a = jnp.where(masks[lvl], x, y)
            b = jnp.where(masks[lvl], y, x)
            return a + shuffle(b, perms[lvl])

        def compute_row(b, slot):
            hc = [h_v[b, pl.ds(c * L, L)] for c in range(ndc)]

            def group(g, carry):
                kb = pl.multiple_of(g * L, L)
                ps = []
                for j in range(L):
                    p = hc[0] * emb[slot, kb + j, pl.ds(0, L)]
                    for c in range(1, ndc):
                        p = p + hc[c] * emb[slot, kb + j, pl.ds(c * L, L)]
                    ps.append(p)
                for lvl in range(4):
                    ps = [combine(ps[2 * i], ps[2 * i + 1], lvl)
                          for i in range(len(ps) // 2)]
                out_v[b, pl.ds(kb, L)] = ps[0]
                return carry

            lax.fori_loop(0, ngroups, group, 0)

        fire(0, 0)

        def body(t, carry):
            b = 2 * t
            fire(b + 1, 1)
            drain(0)
            compute_row(b, 0)

            @pl.when(t < bpw // 2 - 1)
            def _():
                fire(b + 2, 0)

            drain(1)
            compute_row(b + 1, 1)
            return carry

        lax.fori_loop(0, bpw // 2, body, 0)

        pltpu.sync_copy(out_v, out_hbm.at[pl.ds(base, bpw)])

    return sc_kernel


@functools.lru_cache(maxsize=None)
def _build_tc_transpose(D, N):
    assert D == 64
    NB = 1024
    nj = (N + NB - 1) // NB

    def body(x_ref, o_ref):
        t = x_ref[...].T
        o_ref[:, 0:D] = t[0:NB // 2]
        o_ref[:, D:2 * D] = t[NB // 2:]

    return pl.pallas_call(
        body,
        grid=(nj,),
        in_specs=[pl.BlockSpec((D, NB), lambda j: (0, j))],
        out_specs=pl.BlockSpec((NB // 2, 2 * D), lambda j: (j, 0)),
        out_shape=jax.ShapeDtypeStruct((nj * (NB // 2), 2 * D), jnp.float32),
    )


@functools.lru_cache(maxsize=None)
def _build_repack(D, N):
    assert D == 64
    R = 256
    nblk = N // R
    tailr = N - nblk * R
    assert tailr % 8 == 0 and tailr % 2 == 0
    nper = (nblk + NW - 1) // NW
    nper += nper % 2

    mesh = plsc.VectorSubcoreMesh(core_axis_name="c", subcore_axis_name="s")

    @functools.partial(
        pl.kernel,
        mesh=mesh,
        compiler_params=pltpu.CompilerParams(use_tc_tiling_on_sc=True),
        out_type=jax.ShapeDtypeStruct((N // 2, 2 * D), jnp.float32),
        scratch_types=[
            pltpu.VMEM((2, R, D), jnp.float32),
            pltpu.VMEM((2, R // 2, 2 * D), jnp.float32),
            pltpu.SemaphoreType.DMA,
            pltpu.SemaphoreType.DMA,
            pltpu.SemaphoreType.DMA,
            pltpu.SemaphoreType.DMA,
        ],
    )
    def rp_kernel(w_hbm, out_hbm, a_v, b_v, si0, si1, so0, so1):
        wid = lax.axis_index("s") * NC + lax.axis_index("c")
        sin = (si0, si1)
        sout = (so0, so1)

        def blkid(i):
            return wid + i * NW

        def fire_in(i, slot):
            @pl.when(blkid(i) < nblk)
            def _():
                pltpu.async_copy(w_hbm.at[pl.ds(blkid(i) * R, R)],
                                 a_v.at[slot], sin[slot])

        def drain_in(slot):
            pltpu.make_async_copy(w_hbm.at[pl.ds(0, R)], a_v.at[slot],
                                  sin[slot]).wait()

        def fire_out(i, slot):
            pltpu.async_copy(b_v.at[slot],
                             out_hbm.at[pl.ds(blkid(i) * (R // 2), R // 2)],
                             sout[slot])

        def drain_out(slot):
            pltpu.make_async_copy(out_hbm.at[pl.ds(0, R // 2)], b_v.at[slot],
                                  sout[slot]).wait()

        def repack(slot, nm):
            def per_m(m, carry):
                for par in range(2):
                    for c in range(D // L):
                        b_v[slot, m, pl.ds(par * D + c * L, L)] = (
                            a_v[slot, 2 * m + par, pl.ds(c * L, L)])
                return carry

            lax.fori_loop(0, nm, per_m, 0)

        fire_in(0, 0)
        fire_in(1, 1)

        def body(g, carry):
            for slot in (0, 1):
                i = 2 * g + slot

                @pl.when(blkid(i) < nblk)
                def _():
                    drain_in(slot)

                    @pl.when(g >= 1)
                    def _():
                        drain_out(slot)

                    repack(slot, R // 2)
                    fire_out(i, slot)

                fire_in(i + 2, slot)

            return carry

        lax.fori_loop(0, nper // 2, body, 0)
        drain_out(0)
        drain_out(1)

        if tailr:
            @pl.when(wid == 0)
            def _():
                pltpu.sync_copy(w_hbm.at[pl.ds(nblk * R, tailr)],
                                a_v.at[0, pl.ds(0, tailr)])
                repack(0, tailr // 2)
                pltpu.sync_copy(b_v.at[0, pl.ds(0, tailr // 2)],
                                out_hbm.at[pl.ds(nblk * (R // 2), tailr // 2)])

    return rp_kernel


def _remap_ids(ids):
    return ((ids >> 10) << 10) | ((ids & 511) << 1) | ((ids >> 9) & 1)


@functools.lru_cache(maxsize=None)
def _build_transpose(D, N):
    assert D == 64 and 2 * D == 128
    nblk = N // 128
    tailw = N - nblk * 128
    assert tailw in (0, 64)
    nper = (nblk + NW - 1) // NW
    nper += nper % 2

    mesh = plsc.VectorSubcoreMesh(core_axis_name="c", subcore_axis_name="s")

    @functools.partial(
        pl.kernel,
        mesh=mesh,
        compiler_params=pltpu.CompilerParams(use_tc_tiling_on_sc=True),
        out_type=jax.ShapeDtypeStruct((N // 2, 128), jnp.float32),
        scratch_types=[
            pltpu.VMEM((2, D, 128), jnp.float32),
            pltpu.VMEM((2, D, 128), jnp.float32),
            pltpu.VMEM((D, D), jnp.float32),
            pltpu.SemaphoreType.DMA,
            pltpu.SemaphoreType.DMA,
            pltpu.SemaphoreType.DMA,
            pltpu.SemaphoreType.DMA,
        ],
    )
    def tr_kernel(wt_hbm, out_hbm, inb, outb, tailb, si0, si1, so0, so1):
        wid = lax.axis_index("s") * NC + lax.axis_index("c")
        sin = (si0, si1)
        sout = (so0, so1)

        lane = lax.iota(jnp.int32, L)
        dvecs = [16 * j + lane for j in range(D // L)]

        def blkid(i):
            return wid + i * NW

        def fire_in(i, slot):
            @pl.when(blkid(i) < nblk)
            def _():
                pltpu.async_copy(wt_hbm.at[:, pl.ds(blkid(i) * 128, 128)],
                                 inb.at[slot], sin[slot])

        def drain_in(slot):
            pltpu.make_async_copy(wt_hbm.at[:, pl.ds(0, 128)],
                                  inb.at[slot], sin[slot]).wait()

        def fire_out(i, slot):
            pltpu.async_copy(outb.at[slot],
                             out_hbm.at[pl.ds(blkid(i) * D, D)], sout[slot])

        def drain_out(slot):
            pltpu.make_async_copy(out_hbm.at[pl.ds(0, D)],
                                  outb.at[slot], sout[slot]).wait()

        def transpose_rows(src, dst, m):
            for p in range(2):
                cls = jnp.broadcast_to(2 * m + p, (L,))
                for j in range(D // L):
                    dst[m, pl.ds(p * D + 16 * j, L)] = (
                        plsc.load_gather(src, [dvecs[j], cls]))

        def transpose_block(slot, nt):
            def per_m(m, carry):
                transpose_rows(inb.at[slot], outb.at[slot], m)
                return carry

            lax.fori_loop(0, nt * 8, per_m, 0)

        fire_in(0, 0)
        fire_in(1, 1)

        def body(g, carry):
            for slot in (0, 1):
                i = 2 * g + slot
                valid = blkid(i) < nblk

                @pl.when(valid)
                def _():
                    drain_in(slot)

                    @pl.when(g >= 1)
                    def _():
                        drain_out(slot)

                    transpose_block(slot, 8)

                fire_in(i + 2, slot)

                @pl.when(valid)
                def _():
                    fire_out(i, slot)

            return carry

        lax.fori_loop(0, nper // 2, body, 0)
        drain_out(0)
        drain_out(1)

        if tailw:
            @pl.when(wid == 0)
            def _():
                pltpu.sync_copy(wt_hbm.at[:, pl.ds(nblk * 128, tailw)], tailb)

                def per_m(m, carry):
                    transpose_rows(tailb, outb.at[0], m)
                    return carry

                lax.fori_loop(0, tailw // 2, per_m, 0)
                pltpu.sync_copy(outb.at[0, pl.ds(0, tailw // 2)],
                                out_hbm.at[pl.ds(nblk * D, tailw // 2)])

    return tr_kernel


def kernel(h, cand_ids, W):
    B, D = h.shape
    K = cand_ids.shape[1]
    N = W.shape[0]
    ids3 = cand_ids.astype(jnp.int32).reshape(B, 2, K // 2)
    w_rm = _build_repack(D, N)(W).reshape(N, D)
    return _build(B, D, K, N, D)(h, ids3, w_rm)[:, :K]


# speedup vs baseline: 1.2673x; 1.2673x over previous
import functools

import jax
import jax.numpy as jnp
from jax import lax
from jax.experimental import pallas as pl
from jax.experimental.pallas import tpu as pltpu
from jax.experimental.pallas import tpu_sc as plsc

NC = 2
NS = 16
NW = NC * NS
L = 16


@functools.lru_cache(maxsize=None)
def _build(B, D, K, N, DW):
    assert B % NW == 0, B
    assert D % L == 0 and DW >= D, D
    assert K % 2 == 0 and (K // 2) <= 128 and K % 8 == 0 and K >= L, K
    bpw = B // NW
    kh = K // 2
    ngroups = (K + L - 1) // L
    kpad = ngroups * L
    ndc = D // L

    mesh = plsc.VectorSubcoreMesh(core_axis_name="c", subcore_axis_name="s")

    @functools.partial(
        pl.kernel,
        mesh=mesh,
        compiler_params=pltpu.CompilerParams(use_tc_tiling_on_sc=False),
        out_type=jax.ShapeDtypeStruct((B, kpad), jnp.float32),
        scratch_types=[
            pltpu.VMEM((bpw, 2, kh), jnp.int32),
            pltpu.VMEM((bpw, D), jnp.float32),
            pltpu.VMEM((2, kpad, DW), jnp.float32),
            pltpu.VMEM((bpw, kpad), jnp.float32),
            pltpu.SemaphoreType.DMA,
            pltpu.SemaphoreType.DMA,
        ],
    )
    def sc_kernel(h_hbm, ids_hbm, w_hbm, out_hbm, idx_v, h_v, emb, out_v,
                  sem0, sem1):
        wid = lax.axis_index("s") * NC + lax.axis_index("c")
        base = wid * bpw

        pltpu.sync_copy(ids_hbm.at[pl.ds(base, bpw)], idx_v)
        pltpu.sync_copy(h_hbm.at[pl.ds(base, bpw)], h_v)

        sems = (sem0, sem1)

        def fire(b, slot):
            pltpu.async_copy(w_hbm.at[idx_v.at[b, 0]],
                             emb.at[slot, pl.ds(0, kh)], sems[slot])
            pltpu.async_copy(w_hbm.at[idx_v.at[b, 1]],
                             emb.at[slot, pl.ds(kh, kh)], sems[slot])

        def drain(slot):
            pltpu.make_async_copy(w_hbm.at[pl.ds(0, K)],
                                  emb.at[slot, pl.ds(0, K)],
                                  sems[slot]).wait()

        lane = lax.iota(jnp.int32, L)
        perms = [lane ^ (1 << i) for i in range(4)]
        masks = [(lane & (1 << i)) == 0 for i in range(4)]

        dnums = lax.GatherDimensionNumbers(
            offset_dims=(), collapsed_slice_dims=(0,), start_index_map=(0,))

        def shuffle(x, perm):
            return lax.gather(x, perm[:, None], dimension_numbers=dnums,
                              slice_sizes=(1,),
                              mode=lax.GatherScatterMode.PROMISE_IN_BOUNDS)

        def combine(x, y, lvl):
            a = jnp.where(masks[lvl], x, y)
            b = jnp.where(masks[lvl], y, x)
            return a + shuffle(b, perms[lvl])

        def compute_row(b, slot):
            hc = [h_v[b, pl.ds(c * L, L)] for c in range(ndc)]

            def group(g, carry):
                kb = pl.multiple_of(g * L, L)
                ps = []
                for j in range(L):
                    p = hc[0] * emb[slot, kb + j, pl.ds(0, L)]
                    for c in range(1, ndc):
                        p = p + hc[c] * emb[slot, kb + j, pl.ds(c * L, L)]
                    ps.append(p)
                for lvl in range(4):
                    ps = [combine(ps[2 * i], ps[2 * i + 1], lvl)
                          for i in range(len(ps) // 2)]
                out_v[b, pl.ds(kb, L)] = ps[0]
                return carry

            lax.fori_loop(0, ngroups, group, 0)

        fire(0, 0)

        def body(t, carry):
            b = 2 * t
            fire(b + 1, 1)
            drain(0)
            compute_row(b, 0)

            @pl.when(t < bpw // 2 - 1)
            def _():
                fire(b + 2, 0)

            drain(1)
            compute_row(b + 1, 1)
            return carry

        lax.fori_loop(0, bpw // 2, body, 0)

        pltpu.sync_copy(out_v, out_hbm.at[pl.ds(base, bpw)])

    return sc_kernel


def kernel(h, cand_ids, W):
    B, D = h.shape
    K = cand_ids.shape[1]
    N = W.shape[0]
    ids3 = cand_ids.astype(jnp.int32).reshape(B, 2, K // 2)
    wp = jnp.pad(W, ((0, 0), (0, 128 - D)))
    return _build(B, D, K, N, 128)(h, ids3, wp)[:, :K]


# speedup vs baseline: 1.2767x; 1.0074x over previous
import functools

import jax
import jax.numpy as jnp
from jax import lax
from jax.experimental import pallas as pl
from jax.experimental.pallas import tpu as pltpu
from jax.experimental.pallas import tpu_sc as plsc

NC = 2
NS = 16
NW = NC * NS
L = 16


@functools.lru_cache(maxsize=None)
def _build(B, D, K, N, DW):
    assert B % NW == 0, B
    assert D % L == 0 and DW >= D, D
    assert K % 2 == 0 and (K // 2) <= 128 and K % 8 == 0 and K >= L, K
    bpw = B // NW
    kh = K // 2
    ngroups = (K + L - 1) // L
    kpad = ngroups * L
    ndc = D // L

    mesh = plsc.VectorSubcoreMesh(core_axis_name="c", subcore_axis_name="s")

    @functools.partial(
        pl.kernel,
        mesh=mesh,
        compiler_params=pltpu.CompilerParams(use_tc_tiling_on_sc=True),
        out_type=jax.ShapeDtypeStruct((B, kpad), jnp.float32),
        scratch_types=[
            pltpu.VMEM((bpw, 2, kh), jnp.int32),
            pltpu.VMEM((bpw, D), jnp.float32),
            pltpu.VMEM((2, kpad, DW), jnp.float32),
            pltpu.VMEM((2, kpad), jnp.float32),
            pltpu.SemaphoreType.DMA,
            pltpu.SemaphoreType.DMA,
            pltpu.SemaphoreType.DMA,
            pltpu.SemaphoreType.DMA,
        ],
    )
    def sc_kernel(h_hbm, ids_hbm, w_hbm, out_hbm, idx_v, h_v, emb, out_v,
                  sem0, sem1, semo0, semo1):
        wid = lax.axis_index("s") * NC + lax.axis_index("c")
        base = wid * bpw

        pltpu.sync_copy(ids_hbm.at[pl.ds(base, bpw)], idx_v)
        pltpu.sync_copy(h_hbm.at[pl.ds(base, bpw)], h_v)

        sems = (sem0, sem1)
        semos = (semo0, semo1)

        def fire_out(b, slot):
            pltpu.async_copy(out_v.at[slot], out_hbm.at[base + b],
                             semos[slot])

        def drain_out(slot):
            pltpu.make_async_copy(out_hbm.at[0], out_v.at[slot],
                                  semos[slot]).wait()

        def fire(b, slot):
            pltpu.async_copy(w_hbm.at[idx_v.at[b, 0]],
                             emb.at[slot, pl.ds(0, kh)], sems[slot])
            pltpu.async_copy(w_hbm.at[idx_v.at[b, 1]],
                             emb.at[slot, pl.ds(kh, kh)], sems[slot])

        def drain(slot):
            pltpu.make_async_copy(w_hbm.at[pl.ds(0, K)],
                                  emb.at[slot, pl.ds(0, K)],
                                  sems[slot]).wait()

        lane = lax.iota(jnp.int32, L)
        perms = [lane ^ (1 << i) for i in range(4)]
        masks = [(lane & (1 << i)) == 0 for i in range(4)]

        dnums = lax.GatherDimensionNumbers(
            offset_dims=(), collapsed_slice_dims=(0,), start_index_map=(0,))

        def shuffle(x, perm):
            return lax.gather(x, perm[:, None], dimension_numbers=dnums,
                              slice_sizes=(1,),
                              mode=lax.GatherScatterMode.PROMISE_IN_BOUNDS)

        def combine(x, y, lvl):
            a = jnp.where(masks[lvl], x, y)
            b = jnp.where(masks[lvl], y, x)
            return a + shuffle(b, perms[lvl])

        def compute_row(b, slot):
            hc = [h_v[b, pl.ds(c * L, L)] for c in range(ndc)]

            def group(g, carry):
                kb = pl.multiple_of(g * L, L)
                ps = []
                for j in range(L):
                    p = hc[0] * emb[slot, kb + j, pl.ds(0, L)]
                    for c in range(1, ndc):
                        p = p + hc[c] * emb[slot, kb + j, pl.ds(c * L, L)]
                    ps.append(p)
                for lvl in range(4):
                    ps = [combine(ps[2 * i], ps[2 * i + 1], lvl)
                          for i in range(len(ps) // 2)]
                out_v[slot, pl.ds(kb, L)] = ps[0]
                return carry

            lax.fori_loop(0, ngroups, group, 0)

        fire(0, 0)

        def body(t, carry):
            b = 2 * t
            fire(b + 1, 1)
            drain(0)

            @pl.when(t > 0)
            def _():
                drain_out(0)

            compute_row(b, 0)
            fire_out(b, 0)

            @pl.when(t < bpw // 2 - 1)
            def _():
                fire(b + 2, 0)

            drain(1)

            @pl.when(t > 0)
            def _():
                drain_out(1)

            compute_row(b + 1, 1)
            fire_out(b + 1, 1)
            return carry

        lax.fori_loop(0, bpw // 2, body, 0)
        drain_out(0)
        drain_out(1)

    return sc_kernel


def kernel(h, cand_ids, W):
    B, D = h.shape
    K = cand_ids.shape[1]
    N = W.shape[0]
    ids3 = cand_ids.astype(jnp.int32).reshape(B, 2, K // 2)
    wp = jnp.pad(W, ((0, 0), (0, 128 - D)))
    return _build(B, D, K, N, 128)(h, ids3, wp)[:, :K]
